# baseline (speedup 1.0000x reference)
import dataclasses
import functools

import jax
import jax.numpy as jnp
from jax import lax
from jax.experimental import pallas as pl
from jax.experimental.pallas import tpu as pltpu
from jax.experimental.pallas import tpu_sc as plsc

K = 1024
D = 32
N = 65536

BN = 2048

SC_CORES = 2
SC_SUBCORES = 16
NW = SC_CORES * SC_SUBCORES
B_PER_W = N // NW


def _argmin_body(xt_ref, c_ref, ids_ref):
    xt = xt_ref[...]
    c = c_ref[...]
    c2 = jnp.sum(c * c, axis=1, keepdims=True)
    x2 = jnp.sum(xt * xt, axis=0, keepdims=True)
    s = lax.dot_general(
        c.astype(jnp.bfloat16), xt.astype(jnp.bfloat16),
        (((1,), (0,)), ((), ())),
        preferred_element_type=jnp.float32,
    )
    dist = (c2 + x2) - 2.0 * s
    ids_ref[...] = jnp.argmin(dist, axis=0).astype(jnp.int32).reshape(1, 1, BN)


def _compute_ids(inputs_t, clusters, ns, off):
    blk_off = off // BN
    return pl.pallas_call(
        _argmin_body,
        grid=(ns // BN,),
        in_specs=[
            pl.BlockSpec((D, BN), lambda i: (0, i + blk_off)),
            pl.BlockSpec((K, D), lambda i: (0, 0)),
        ],
        out_specs=pl.BlockSpec((1, 1, BN), lambda i: (i, 0, 0)),
        out_shape=jax.ShapeDtypeStruct((ns // BN, 1, BN), jnp.int32),
    )(inputs_t, clusters)


PADW = 128
CHUNK = 256
NCHUNK = B_PER_W // CHUNK


def _sc_gather(table_pad, idx, ns):
    b_per_w = ns // NW
    nchunk = b_per_w // CHUNK
    mesh = plsc.VectorSubcoreMesh(core_axis_name="c", subcore_axis_name="s")
    cp = pltpu.CompilerParams()
    if "needs_layout_passes" in pltpu.CompilerParams.__dataclass_fields__:
        cp = dataclasses.replace(cp, needs_layout_passes=False)

    @functools.partial(
        pl.kernel,
        mesh=mesh,
        compiler_params=cp,
        out_type=jax.ShapeDtypeStruct((ns, PADW), jnp.float32),
        scratch_types=[
            pltpu.VMEM((b_per_w,), jnp.int32),
            pltpu.VMEM((CHUNK, PADW), jnp.float32),
            pltpu.VMEM((CHUNK, PADW), jnp.float32),
            pltpu.SemaphoreType.DMA,
            pltpu.SemaphoreType.DMA,
        ],
    )
    def k(table_hbm, idx_hbm, out_hbm, idx_v, rows_a, rows_b,
          sem_a, sem_b):
        wid = lax.axis_index("s") * SC_CORES + lax.axis_index("c")
        base = wid * b_per_w
        pltpu.sync_copy(idx_hbm.at[pl.ds(base, b_per_w)], idx_v)

        bufs = (rows_a, rows_b)
        sems = (sem_a, sem_b)

        def gather(c):
            return pltpu.async_copy(
                table_hbm.at[idx_v.at[pl.ds(c * CHUNK, CHUNK)]],
                bufs[c % 2], sems[c % 2])

        cps = {0: gather(0)}
        for c in range(nchunk):
            if c + 1 < nchunk:
                cps[c + 1] = gather(c + 1)
            cps[c].wait()
            pltpu.sync_copy(
                bufs[c % 2], out_hbm.at[pl.ds(base + c * CHUNK, CHUNK), :])

    return k(table_pad, idx)


NSLICE = 8


def kernel(inputs, clusters):
    xt = inputs.T
    table_pad = jnp.pad(clusters, ((0, 0), (0, PADW - D)))
    ns = N // NSLICE
    ids_parts, cent_parts = [], []
    for si in range(NSLICE):
        ids_s = _compute_ids(xt, clusters, ns, si * ns).reshape(ns)
        cents_s = _sc_gather(table_pad, ids_s, ns)[:, :D]
        ids_parts.append(ids_s)
        cent_parts.append(cents_s)
    return (jnp.concatenate(ids_parts),
            jnp.concatenate(cent_parts, axis=0))


# speedup vs baseline: 1.0556x; 1.0556x over previous
import dataclasses
import functools

import jax
import jax.numpy as jnp
from jax import lax
from jax.experimental import pallas as pl
from jax.experimental.pallas import tpu as pltpu
from jax.experimental.pallas import tpu_sc as plsc

K = 1024
D = 32
N = 65536

BN = 2048

SC_CORES = 2
SC_SUBCORES = 16
NW = SC_CORES * SC_SUBCORES
B_PER_W = N // NW


def _argmin_body(xt_ref, c_ref, ids_ref):
    xt = xt_ref[...]
    c = c_ref[...]
    c2 = jnp.sum(c * c, axis=1, keepdims=True)
    x2 = jnp.sum(xt * xt, axis=0, keepdims=True)
    s = lax.dot_general(
        c.astype(jnp.bfloat16), xt.astype(jnp.bfloat16),
        (((1,), (0,)), ((), ())),
        preferred_element_type=jnp.float32,
    )
    dist = (c2 + x2) - 2.0 * s
    ids_ref[...] = jnp.argmin(dist, axis=0).astype(jnp.int32).reshape(1, 1, BN)


def _compute_ids(inputs_t, clusters, ns, off):
    blk_off = off // BN
    return pl.pallas_call(
        _argmin_body,
        grid=(ns // BN,),
        in_specs=[
            pl.BlockSpec((D, BN), lambda i: (0, i + blk_off)),
            pl.BlockSpec((K, D), lambda i: (0, 0)),
        ],
        out_specs=pl.BlockSpec((1, 1, BN), lambda i: (i, 0, 0)),
        out_shape=jax.ShapeDtypeStruct((ns // BN, 1, BN), jnp.int32),
    )(inputs_t, clusters)


PADW = 128
CHUNK = 256
NCHUNK = B_PER_W // CHUNK


def _sc_gather(table_pad, idx, ns):
    b_per_w = ns // NW
    nchunk = b_per_w // CHUNK
    mesh = plsc.VectorSubcoreMesh(core_axis_name="c", subcore_axis_name="s")
    cp = pltpu.CompilerParams()
    if "needs_layout_passes" in pltpu.CompilerParams.__dataclass_fields__:
        cp = dataclasses.replace(cp, needs_layout_passes=False)

    @functools.partial(
        pl.kernel,
        mesh=mesh,
        compiler_params=cp,
        out_type=jax.ShapeDtypeStruct((ns, PADW), jnp.float32),
        scratch_types=[
            pltpu.VMEM((b_per_w,), jnp.int32),
            pltpu.VMEM((CHUNK, PADW), jnp.float32),
            pltpu.VMEM((CHUNK, PADW), jnp.float32),
            pltpu.SemaphoreType.DMA,
            pltpu.SemaphoreType.DMA,
        ],
    )
    def k(table_hbm, idx_hbm, out_hbm, idx_v, rows_a, rows_b,
          sem_a, sem_b):
        wid = lax.axis_index("s") * SC_CORES + lax.axis_index("c")
        base = wid * b_per_w
        pltpu.sync_copy(idx_hbm.at[pl.ds(base, b_per_w)], idx_v)

        bufs = (rows_a, rows_b)
        sems = (sem_a, sem_b)

        def gather(c):
            return pltpu.async_copy(
                table_hbm.at[idx_v.at[pl.ds(c * CHUNK, CHUNK)]],
                bufs[c % 2], sems[c % 2])

        cps = {0: gather(0)}
        for c in range(nchunk):
            if c + 1 < nchunk:
                cps[c + 1] = gather(c + 1)
            cps[c].wait()
            pltpu.sync_copy(
                bufs[c % 2], out_hbm.at[pl.ds(base + c * CHUNK, CHUNK), :])

    return k(table_pad, idx)


NSLICE = 4


def kernel(inputs, clusters):
    xt = inputs.T
    table_pad = jnp.pad(clusters, ((0, 0), (0, PADW - D)))
    ns = N // NSLICE
    ids_parts, cent_parts = [], []
    for si in range(NSLICE):
        ids_s = _compute_ids(xt, clusters, ns, si * ns).reshape(ns)
        cents_s = _sc_gather(table_pad, ids_s, ns)[:, :D]
        ids_parts.append(ids_s)
        cent_parts.append(cents_s)
    return (jnp.concatenate(ids_parts),
            jnp.concatenate(cent_parts, axis=0))
